# baseline (speedup 1.0000x reference)
import functools

import jax
import jax.numpy as jnp
from jax import lax
from jax.experimental import pallas as pl
from jax.experimental.pallas import tpu as pltpu
from jax.experimental.pallas import tpu_sc as plsc

D = 768
E = 8
F = 3072
TOPK = 2
BT = 256
NW = 32

def _sc_mesh():
    return plsc.VectorSubcoreMesh(core_axis_name="c", subcore_axis_name="s",
                                  num_cores=2, num_subcores=16)


D2 = D // 2


def _pack_bf16_pair(y):
    a = lax.bitcast_convert_type(y[:, :D2], jnp.int32)
    b = lax.bitcast_convert_type(y[:, D2:], jnp.int32)
    ra = (a + 0x7FFF + ((a >> 16) & 1)) >> 16
    rb = (b + 0x7FFF + ((b >> 16) & 1)) >> 16
    return (ra & 0xFFFF) | (rb << 16)


def _unpack_halves_f32(p):
    lo = lax.bitcast_convert_type(p << 16, jnp.float32)
    hi = lax.bitcast_convert_type(p & jnp.int32(-65536), jnp.float32)
    return lo, hi


def _router_body(x_ref, Wr_ref, br_ref, e0_ref, e1_ref, g0_ref, g1_ref,
                 xp_ref):
    x = x_ref[0]
    xp_ref[...] = _pack_bf16_pair(x)
    m = jnp.dot(x, Wr_ref[...],
                preferred_element_type=jnp.float32) + br_ref[...]
    s = m.shape[0]
    ii = lax.broadcasted_iota(jnp.int32, (s, E), 1)
    v0 = jnp.max(m, axis=1, keepdims=True)
    e0 = jnp.min(jnp.where(m == v0, ii, E), axis=1, keepdims=True)
    m1 = jnp.where(ii == e0, -1e30, m)
    v1 = jnp.max(m1, axis=1, keepdims=True)
    e1 = jnp.min(jnp.where(m1 == v1, ii, E), axis=1, keepdims=True)
    r = jnp.exp(v1 - v0)
    z = 1.0 + r
    e0_ref[...] = e0
    e1_ref[...] = e1
    g0_ref[...] = 1.0 / z
    g1_ref[...] = r / z


def _route(x3, Wr, br):
    s = x3.shape[1]
    return pl.pallas_call(
        _router_body,
        out_shape=(
            jax.ShapeDtypeStruct((s, 1), jnp.int32),
            jax.ShapeDtypeStruct((s, 1), jnp.int32),
            jax.ShapeDtypeStruct((s, 1), jnp.float32),
            jax.ShapeDtypeStruct((s, 1), jnp.float32),
            jax.ShapeDtypeStruct((s, D2), jnp.int32),
        ),
    )(x3, Wr, br.reshape(1, E))


def _make_sc_dispatch(npair, npad):
    ppw = npair // NW
    nc = 2
    cs = ppw // nc

    s = npair // TOPK

    @functools.partial(
        pl.kernel,
        out_type=jax.ShapeDtypeStruct((npad, D2), jnp.int32),
        mesh=_sc_mesh(),
        scratch_types=[
            [pltpu.VMEM((cs,), jnp.int32) for _ in range(nc)],
            [pltpu.VMEM((cs, D2), jnp.int32) for _ in range(nc)],
            [pltpu.SemaphoreType.DMA for _ in range(nc)],
            [pltpu.SemaphoreType.DMA for _ in range(nc)],
        ],
    )
    def dispatch(x_hbm, dest_hbm, xs_hbm, dest_v, rows_v, sem_g, sem_s):
        wid = lax.axis_index("s") * 2 + lax.axis_index("c")
        base = wid * ppw
        gathers = []
        for k in range(nc):
            pltpu.sync_copy(dest_hbm.at[pl.ds(base + k * cs, cs)], dest_v[k])
            row0 = lax.rem(base + k * cs, s)
            gathers.append(pltpu.async_copy(
                x_hbm.at[pl.ds(row0, cs)], rows_v[k], sem_g[k]))
        scatters = []
        for k in range(nc):
            gathers[k].wait()
            scatters.append(
                pltpu.async_copy(rows_v[k], xs_hbm.at[dest_v[k]], sem_s[k]))
        for c in scatters:
            c.wait()

    return dispatch


def _make_sc_combine(npair, npad):
    ppw = npair // NW
    nc = 2
    cs = ppw // nc

    @functools.partial(
        pl.kernel,
        out_type=jax.ShapeDtypeStruct((npair, D2), jnp.int32),
        mesh=_sc_mesh(),
        scratch_types=[
            [pltpu.VMEM((cs,), jnp.int32) for _ in range(nc)],
            [pltpu.VMEM((cs, D2), jnp.int32) for _ in range(nc)],
            [pltpu.SemaphoreType.DMA for _ in range(nc)],
            [pltpu.SemaphoreType.DMA for _ in range(nc)],
        ],
    )
    def combine(ys_hbm, dest_hbm, a_hbm, dest_v, rows_v, sem_g, sem_s):
        wid = lax.axis_index("s") * 2 + lax.axis_index("c")
        base = wid * ppw
        gathers = []
        for k in range(nc):
            pltpu.sync_copy(dest_hbm.at[pl.ds(base + k * cs, cs)], dest_v[k])
            gathers.append(
                pltpu.async_copy(ys_hbm.at[dest_v[k]], rows_v[k], sem_g[k]))
        writes = []
        for k in range(nc):
            gathers[k].wait()
            writes.append(pltpu.async_copy(
                rows_v[k], a_hbm.at[pl.ds(base + k * cs, cs)], sem_s[k]))
        for c in writes:
            c.wait()

    return combine


def _ffn_body(be_ref, bv_ref, xs_ref, W1_ref, b1_ref, W2_ref, b2_ref, ys_ref):
    b = pl.program_id(0)

    @pl.when(bv_ref[b] != 0)
    def _():
        lo, hi = _unpack_halves_f32(xs_ref[...])
        xb = jnp.concatenate([lo, hi], axis=1).astype(jnp.bfloat16)
        h = jnp.dot(xb, W1_ref[0].astype(jnp.bfloat16),
                    preferred_element_type=jnp.float32)
        h = jnp.maximum(h + b1_ref[0], 0.0)
        y = jnp.dot(h.astype(jnp.bfloat16), W2_ref[0].astype(jnp.bfloat16),
                    preferred_element_type=jnp.float32)
        ys_ref[...] = _pack_bf16_pair(y + b2_ref[0])


def _grouped_ffn(xs, W1, b1, W2, b2, block_expert, block_valid, nb):
    grid_spec = pltpu.PrefetchScalarGridSpec(
        num_scalar_prefetch=2,
        grid=(nb,),
        in_specs=[
            pl.BlockSpec((BT, D2), lambda b, be, bv: (b * bv[b], 0)),
            pl.BlockSpec((1, D, F), lambda b, be, bv: (be[b], 0, 0)),
            pl.BlockSpec((1, 1, F), lambda b, be, bv: (be[b], 0, 0)),
            pl.BlockSpec((1, F, D), lambda b, be, bv: (be[b], 0, 0)),
            pl.BlockSpec((1, 1, D), lambda b, be, bv: (be[b], 0, 0)),
        ],
        out_specs=pl.BlockSpec(
            (BT, D2), lambda b, be, bv: (b * bv[b] + (1 - bv[b]) * nb, 0)),
    )
    return pl.pallas_call(
        _ffn_body,
        grid_spec=grid_spec,
        out_shape=jax.ShapeDtypeStruct((xs.shape[0] + BT, D2), jnp.int32),
        compiler_params=pltpu.CompilerParams(
            dimension_semantics=("arbitrary",)),
    )(block_expert, block_valid, xs, W1, b1.reshape(E, 1, F), W2,
      b2.reshape(E, 1, D))


def _mix_body(a_ref, b_ref, g0_ref, g1_ref, out_ref):
    alo, ahi = _unpack_halves_f32(a_ref[...])
    blo, bhi = _unpack_halves_f32(b_ref[...])
    g0 = g0_ref[...]
    g1 = g1_ref[...]
    out_ref[0, :, :D2] = g0 * alo + g1 * blo
    out_ref[0, :, D2:] = g0 * ahi + g1 * bhi


def _mix(a, g0, g1):
    s = g0.shape[0]
    bt = 512
    nblk = s // bt
    return pl.pallas_call(
        _mix_body,
        grid=(nblk,),
        in_specs=[
            pl.BlockSpec((bt, D2), lambda t: (t, 0)),
            pl.BlockSpec((bt, D2), lambda t, _n=nblk: (t + _n, 0)),
            pl.BlockSpec((bt, 1), lambda t: (t, 0)),
            pl.BlockSpec((bt, 1), lambda t: (t, 0)),
        ],
        out_specs=pl.BlockSpec((1, bt, D), lambda t: (0, t, 0)),
        out_shape=jax.ShapeDtypeStruct((1, s, D), jnp.float32),
    )(a, a, g0, g1)


def kernel(x, Wr, br, W1, b1, W2, b2):
    B, S, D_ = x.shape
    npair = S * TOPK
    nb = npair // BT + E
    npad = nb * BT

    e0, e1, g0, g1, xp = _route(x, Wr, br)

    e_flat = jnp.concatenate([e0, e1], axis=0).reshape(-1)
    onehf = (e_flat[:, None] == jnp.arange(E, dtype=jnp.int32)[None, :]
             ).astype(jnp.float32)
    ranks = jnp.cumsum(onehf, axis=0) - onehf
    rank = jnp.sum(ranks * onehf, axis=1)
    counts = jnp.sum(onehf, axis=0).astype(jnp.int32)
    padded = ((counts + BT - 1) // BT) * BT
    pend = jnp.cumsum(padded)
    pstart = (pend - padded).astype(jnp.float32)
    dest = (jnp.sum(onehf * pstart[None, :], axis=1) + rank).astype(jnp.int32)
    bpos = jnp.arange(nb, dtype=jnp.int32) * BT
    block_expert = jnp.minimum(
        jnp.sum((pend[None, :] <= bpos[:, None]).astype(jnp.int32), axis=1),
        E - 1)
    block_valid = (bpos < pend[-1]).astype(jnp.int32)

    xs = _make_sc_dispatch(npair, npad)(xp, dest)

    ys = _grouped_ffn(xs, W1, b1, W2, b2, block_expert, block_valid, nb)

    a = _make_sc_combine(npair, npad)(ys, dest)

    return _mix(a, g0, g1)


# speedup vs baseline: 1.0880x; 1.0880x over previous
import functools

import jax
import jax.numpy as jnp
from jax import lax
from jax.experimental import pallas as pl
from jax.experimental.pallas import tpu as pltpu
from jax.experimental.pallas import tpu_sc as plsc

D = 768
E = 8
F = 3072
TOPK = 2
BT = 512
NW = 32

def _sc_mesh():
    return plsc.VectorSubcoreMesh(core_axis_name="c", subcore_axis_name="s",
                                  num_cores=2, num_subcores=16)


D2 = D // 2


def _pack_bf16_pair(y):
    a = lax.bitcast_convert_type(y[:, :D2], jnp.int32)
    b = lax.bitcast_convert_type(y[:, D2:], jnp.int32)
    ra = (a + 0x7FFF + ((a >> 16) & 1)) >> 16
    rb = (b + 0x7FFF + ((b >> 16) & 1)) >> 16
    return (ra & 0xFFFF) | (rb << 16)


def _unpack_halves_f32(p):
    lo = lax.bitcast_convert_type(p << 16, jnp.float32)
    hi = lax.bitcast_convert_type(p & jnp.int32(-65536), jnp.float32)
    return lo, hi


def _router_body(x_ref, Wr_ref, br_ref, e0_ref, e1_ref, g0_ref, g1_ref,
                 xp_ref):
    x = x_ref[0]
    xp_ref[...] = _pack_bf16_pair(x)
    m = jnp.dot(x, Wr_ref[...],
                preferred_element_type=jnp.float32) + br_ref[...]
    s = m.shape[0]
    ii = lax.broadcasted_iota(jnp.int32, (s, E), 1)
    v0 = jnp.max(m, axis=1, keepdims=True)
    e0 = jnp.min(jnp.where(m == v0, ii, E), axis=1, keepdims=True)
    m1 = jnp.where(ii == e0, -1e30, m)
    v1 = jnp.max(m1, axis=1, keepdims=True)
    e1 = jnp.min(jnp.where(m1 == v1, ii, E), axis=1, keepdims=True)
    r = jnp.exp(v1 - v0)
    z = 1.0 + r
    e0_ref[...] = e0
    e1_ref[...] = e1
    g0_ref[...] = 1.0 / z
    g1_ref[...] = r / z


def _route(x3, Wr, br):
    s = x3.shape[1]
    return pl.pallas_call(
        _router_body,
        out_shape=(
            jax.ShapeDtypeStruct((s, 1), jnp.int32),
            jax.ShapeDtypeStruct((s, 1), jnp.int32),
            jax.ShapeDtypeStruct((s, 1), jnp.float32),
            jax.ShapeDtypeStruct((s, 1), jnp.float32),
            jax.ShapeDtypeStruct((s, D2), jnp.int32),
        ),
    )(x3, Wr, br.reshape(1, E))


def _make_sc_dispatch(npair, npad):
    ppw = npair // NW
    nc = 2
    cs = ppw // nc

    s = npair // TOPK

    @functools.partial(
        pl.kernel,
        out_type=jax.ShapeDtypeStruct((npad, D2), jnp.int32),
        mesh=_sc_mesh(),
        scratch_types=[
            [pltpu.VMEM((cs,), jnp.int32) for _ in range(nc)],
            [pltpu.VMEM((cs, D2), jnp.int32) for _ in range(nc)],
            [pltpu.SemaphoreType.DMA for _ in range(nc)],
            [pltpu.SemaphoreType.DMA for _ in range(nc)],
        ],
    )
    def dispatch(x_hbm, dest_hbm, xs_hbm, dest_v, rows_v, sem_g, sem_s):
        wid = lax.axis_index("s") * 2 + lax.axis_index("c")
        base = wid * ppw
        gathers = []
        for k in range(nc):
            pltpu.sync_copy(dest_hbm.at[pl.ds(base + k * cs, cs)], dest_v[k])
            row0 = lax.rem(base + k * cs, s)
            gathers.append(pltpu.async_copy(
                x_hbm.at[pl.ds(row0, cs)], rows_v[k], sem_g[k]))
        scatters = []
        for k in range(nc):
            gathers[k].wait()
            scatters.append(
                pltpu.async_copy(rows_v[k], xs_hbm.at[dest_v[k]], sem_s[k]))
        for c in scatters:
            c.wait()

    return dispatch


def _make_sc_combine(npair, npad):
    ppw = npair // NW
    nc = 2
    cs = ppw // nc

    @functools.partial(
        pl.kernel,
        out_type=jax.ShapeDtypeStruct((npair, D2), jnp.int32),
        mesh=_sc_mesh(),
        scratch_types=[
            [pltpu.VMEM((cs,), jnp.int32) for _ in range(nc)],
            [pltpu.VMEM((cs, D2), jnp.int32) for _ in range(nc)],
            [pltpu.SemaphoreType.DMA for _ in range(nc)],
            [pltpu.SemaphoreType.DMA for _ in range(nc)],
        ],
    )
    def combine(ys_hbm, dest_hbm, a_hbm, dest_v, rows_v, sem_g, sem_s):
        wid = lax.axis_index("s") * 2 + lax.axis_index("c")
        base = wid * ppw
        gathers = []
        for k in range(nc):
            pltpu.sync_copy(dest_hbm.at[pl.ds(base + k * cs, cs)], dest_v[k])
            gathers.append(
                pltpu.async_copy(ys_hbm.at[dest_v[k]], rows_v[k], sem_g[k]))
        writes = []
        for k in range(nc):
            gathers[k].wait()
            writes.append(pltpu.async_copy(
                rows_v[k], a_hbm.at[pl.ds(base + k * cs, cs)], sem_s[k]))
        for c in writes:
            c.wait()

    return combine


def _ffn_body(be_ref, bv_ref, xs_ref, W1_ref, b1_ref, W2_ref, b2_ref, ys_ref):
    b = pl.program_id(0)

    @pl.when(bv_ref[b] != 0)
    def _():
        lo, hi = _unpack_halves_f32(xs_ref[...])
        xb = jnp.concatenate([lo, hi], axis=1).astype(jnp.bfloat16)
        h = jnp.dot(xb, W1_ref[0].astype(jnp.bfloat16),
                    preferred_element_type=jnp.float32)
        h = jnp.maximum(h + b1_ref[0], 0.0)
        y = jnp.dot(h.astype(jnp.bfloat16), W2_ref[0].astype(jnp.bfloat16),
                    preferred_element_type=jnp.float32)
        ys_ref[...] = _pack_bf16_pair(y + b2_ref[0])


def _grouped_ffn(xs, W1, b1, W2, b2, block_expert, block_valid, nb):
    grid_spec = pltpu.PrefetchScalarGridSpec(
        num_scalar_prefetch=2,
        grid=(nb,),
        in_specs=[
            pl.BlockSpec((BT, D2), lambda b, be, bv: (b * bv[b], 0)),
            pl.BlockSpec((1, D, F), lambda b, be, bv: (be[b], 0, 0)),
            pl.BlockSpec((1, 1, F), lambda b, be, bv: (be[b], 0, 0)),
            pl.BlockSpec((1, F, D), lambda b, be, bv: (be[b], 0, 0)),
            pl.BlockSpec((1, 1, D), lambda b, be, bv: (be[b], 0, 0)),
        ],
        out_specs=pl.BlockSpec(
            (BT, D2), lambda b, be, bv: (b * bv[b] + (1 - bv[b]) * nb, 0)),
    )
    return pl.pallas_call(
        _ffn_body,
        grid_spec=grid_spec,
        out_shape=jax.ShapeDtypeStruct((xs.shape[0] + BT, D2), jnp.int32),
        compiler_params=pltpu.CompilerParams(
            dimension_semantics=("arbitrary",)),
    )(block_expert, block_valid, xs, W1, b1.reshape(E, 1, F), W2,
      b2.reshape(E, 1, D))


def _mix_body(a_ref, b_ref, g0_ref, g1_ref, out_ref):
    alo, ahi = _unpack_halves_f32(a_ref[...])
    blo, bhi = _unpack_halves_f32(b_ref[...])
    g0 = g0_ref[...]
    g1 = g1_ref[...]
    out_ref[0, :, :D2] = g0 * alo + g1 * blo
    out_ref[0, :, D2:] = g0 * ahi + g1 * bhi


def _mix(a, g0, g1):
    s = g0.shape[0]
    bt = 512
    nblk = s // bt
    return pl.pallas_call(
        _mix_body,
        grid=(nblk,),
        in_specs=[
            pl.BlockSpec((bt, D2), lambda t: (t, 0)),
            pl.BlockSpec((bt, D2), lambda t, _n=nblk: (t + _n, 0)),
            pl.BlockSpec((bt, 1), lambda t: (t, 0)),
            pl.BlockSpec((bt, 1), lambda t: (t, 0)),
        ],
        out_specs=pl.BlockSpec((1, bt, D), lambda t: (0, t, 0)),
        out_shape=jax.ShapeDtypeStruct((1, s, D), jnp.float32),
    )(a, a, g0, g1)


def kernel(x, Wr, br, W1, b1, W2, b2):
    B, S, D_ = x.shape
    npair = S * TOPK
    nb = npair // BT + E
    npad = nb * BT

    e0, e1, g0, g1, xp = _route(x, Wr, br)

    e_flat = jnp.concatenate([e0, e1], axis=0).reshape(-1)
    onehf = (e_flat[:, None] == jnp.arange(E, dtype=jnp.int32)[None, :]
             ).astype(jnp.float32)
    ranks = jnp.cumsum(onehf, axis=0) - onehf
    rank = jnp.sum(ranks * onehf, axis=1)
    counts = jnp.sum(onehf, axis=0).astype(jnp.int32)
    padded = ((counts + BT - 1) // BT) * BT
    pend = jnp.cumsum(padded)
    pstart = (pend - padded).astype(jnp.float32)
    dest = (jnp.sum(onehf * pstart[None, :], axis=1) + rank).astype(jnp.int32)
    bpos = jnp.arange(nb, dtype=jnp.int32) * BT
    block_expert = jnp.minimum(
        jnp.sum((pend[None, :] <= bpos[:, None]).astype(jnp.int32), axis=1),
        E - 1)
    block_valid = (bpos < pend[-1]).astype(jnp.int32)

    xs = _make_sc_dispatch(npair, npad)(xp, dest)

    ys = _grouped_ffn(xs, W1, b1, W2, b2, block_expert, block_valid, nb)

    a = _make_sc_combine(npair, npad)(ys, dest)

    return _mix(a, g0, g1)
